# baseline (speedup 1.0000x reference)
import functools

import jax
import jax.numpy as jnp
import numpy as np
from jax import lax
from jax.experimental import pallas as pl
from jax.experimental.pallas import tpu as pltpu
from jax.experimental.pallas import tpu_sc as plsc

T = 8192
D = 768
H = 768
O = 768
E = 8
K_ACTIVE = 5

_INV_TEMP = float(1.0 / np.e)


_TT_ROUTE = 2048


def _routing_scores_body(x_ref, gw_ref, gb_ref, w1_ref, b1_ref, w2_ref, b2_ref,
                         sigt_ref, out_ref):
    x = x_ref[...]
    scores = (jnp.dot(x, gw_ref[...], preferred_element_type=jnp.float32)
              + gb_ref[...]) * _INV_TEMP
    scores = scores - jnp.max(scores, axis=-1, keepdims=True)
    exps = jnp.exp(scores)
    probs = exps / jnp.sum(exps, axis=-1, keepdims=True)

    h1 = jnp.maximum(
        jnp.dot(x, w1_ref[...], preferred_element_type=jnp.float32)
        + b1_ref[...], 0.0)
    ph = jnp.dot(h1, w2_ref[...], preferred_element_type=jnp.float32) + b2_ref[...]
    ph_norm = ph / jnp.maximum(
        jnp.sqrt(jnp.sum(ph * ph, axis=-1, keepdims=True)), 1e-12)

    sigt = sigt_ref[...]
    sig_norm = sigt / jnp.maximum(
        jnp.sqrt(jnp.sum(sigt * sigt, axis=0, keepdims=True)), 1e-12)
    ph_match = (jnp.dot(ph_norm, sig_norm, preferred_element_type=jnp.float32)
                + 1.0) * 0.5
    out_ref[...] = probs * ph_match


def _routing_scores(x, gate_W, gate_b, ph_W1, ph_b1, ph_W2, ph_b2, sigs):
    full = lambda shape: pl.BlockSpec(shape, lambda i: (0,) * len(shape))
    return pl.pallas_call(
        _routing_scores_body,
        grid=(T // _TT_ROUTE,),
        in_specs=[
            pl.BlockSpec((_TT_ROUTE, D), lambda i: (i, 0)),
            full((D, E)),
            full((1, E)),
            full((D, 32)),
            full((1, 32)),
            full((32, 16)),
            full((1, 16)),
            full((16, E)),
        ],
        out_specs=pl.BlockSpec((_TT_ROUTE, E), lambda i: (i, 0)),
        out_shape=jax.ShapeDtypeStruct((T, E), jnp.float32),
        compiler_params=pltpu.CompilerParams(
            dimension_semantics=("parallel",)),
    )(x, gate_W, gate_b.reshape(1, E), ph_W1, ph_b1.reshape(1, 32), ph_W2,
      ph_b2.reshape(1, 16), sigs.T)


_NC = 2
_NS = 16
_NW = _NC * _NS
_TOK_PER_W = T // _NW
_VALS_PER_W = _TOK_PER_W * E
_LANES = 16
_CHUNKS = _TOK_PER_W // _LANES


def _topk_normalize_body(eff_hbm, out_hbm, vin, vout):
    wid = lax.axis_index("s") * _NC + lax.axis_index("c")
    base = wid * _VALS_PER_W
    pltpu.sync_copy(eff_hbm.at[pl.ds(base, _VALS_PER_W)], vin)

    def chunk(c, carry):
        lane = lax.iota(jnp.int32, _LANES)
        tok = c * _LANES + lane
        idx = [tok * E + e for e in range(E)]
        v = [plsc.load_gather(vin, [idx[e]]) for e in range(E)]

        total = jnp.zeros((_LANES,), jnp.float32)
        w = []
        for e in range(E):
            rank = jnp.zeros((_LANES,), jnp.float32)
            for j in range(E):
                if j == e:
                    continue
                beats = (v[j] >= v[e]) if j < e else (v[j] > v[e])
                rank = rank + jnp.where(beats, 1.0, 0.0)
            we = jnp.where(rank < float(K_ACTIVE), v[e], 0.0)
            total = total + we
            w.append(we)
        inv = 1.0 / (total + 1e-8)
        for e in range(E):
            plsc.store_scatter(vout, [idx[e]], w[e] * inv)
        return carry

    lax.fori_loop(0, _CHUNKS, chunk, 0)
    pltpu.sync_copy(vout, out_hbm.at[pl.ds(base, _VALS_PER_W)])


def _topk_normalize(effective):
    kern = functools.partial(
        pl.kernel,
        out_type=jax.ShapeDtypeStruct((T * E,), jnp.float32),
        mesh=plsc.VectorSubcoreMesh(core_axis_name="c", subcore_axis_name="s"),
        scratch_types=[
            pltpu.VMEM((_VALS_PER_W,), jnp.float32),
            pltpu.VMEM((_VALS_PER_W,), jnp.float32),
        ],
        compiler_params=pltpu.CompilerParams(needs_layout_passes=False),
    )(_topk_normalize_body)
    return kern(effective.reshape(T * E)).reshape(T, E)


_TT_EXP = 512


def _experts_body(x_ref, w_ref, w1_ref, b1_ref, w2_ref, b2_ref, out_ref):
    x = x_ref[...].astype(jnp.bfloat16)
    w = w_ref[...]
    lane = lax.broadcasted_iota(jnp.int32, (_TT_EXP, E), 1)
    acc = None
    for e in range(E):
        h = jnp.maximum(
            jnp.dot(x, w1_ref[e], preferred_element_type=jnp.float32)
            + b1_ref[e, 0], 0.0)
        y = (jnp.dot(h.astype(jnp.bfloat16), w2_ref[e],
                     preferred_element_type=jnp.float32)
             + b2_ref[e, 0])
        wsel = jnp.sum(jnp.where(lane == e, w, 0.0), axis=1, keepdims=True)
        contrib = y * wsel
        acc = contrib if acc is None else acc + contrib
    out_ref[...] = acc


def _experts_combine(x, weights, eW1, eb1, eW2, eb2):
    return pl.pallas_call(
        _experts_body,
        grid=(T // _TT_EXP,),
        in_specs=[
            pl.BlockSpec((_TT_EXP, D), lambda i: (i, 0)),
            pl.BlockSpec((_TT_EXP, E), lambda i: (i, 0)),
            pl.BlockSpec((E, D, H), lambda i: (0, 0, 0)),
            pl.BlockSpec((E, 1, H), lambda i: (0, 0, 0)),
            pl.BlockSpec((E, H, O), lambda i: (0, 0, 0)),
            pl.BlockSpec((E, 1, O), lambda i: (0, 0, 0)),
        ],
        out_specs=pl.BlockSpec((_TT_EXP, O), lambda i: (i, 0)),
        out_shape=jax.ShapeDtypeStruct((T, O), jnp.float32),
        compiler_params=pltpu.CompilerParams(
            dimension_semantics=("arbitrary",)),
    )(x, weights, eW1.astype(jnp.bfloat16), eb1.reshape(E, 1, H),
      eW2.astype(jnp.bfloat16), eb2.reshape(E, 1, O))


@jax.jit
def kernel(x, gate_W, gate_b, ph_W1, ph_b1, ph_W2, ph_b2, sigs, eW1, eb1,
           eW2, eb2):
    effective = _routing_scores(x, gate_W, gate_b, ph_W1, ph_b1, ph_W2, ph_b2,
                                sigs)
    weights = _topk_normalize(effective)
    return _experts_combine(x, weights, eW1, eb1, eW2, eb2)


# speedup vs baseline: 1.0684x; 1.0684x over previous
import functools

import jax
import jax.numpy as jnp
import numpy as np
from jax import lax
from jax.experimental import pallas as pl
from jax.experimental.pallas import tpu as pltpu
from jax.experimental.pallas import tpu_sc as plsc

T = 8192
D = 768
H = 768
O = 768
E = 8
K_ACTIVE = 5

_INV_TEMP = float(1.0 / np.e)


_TT_ROUTE = 2048


def _routing_scores_body(x_ref, gw_ref, gb_ref, w1_ref, b1_ref, w2_ref, b2_ref,
                         sigt_ref, out_ref):
    x = x_ref[...]
    scores = (jnp.dot(x, gw_ref[...], preferred_element_type=jnp.float32)
              + gb_ref[...]) * _INV_TEMP
    scores = scores - jnp.max(scores, axis=-1, keepdims=True)
    exps = jnp.exp(scores)
    probs = exps / jnp.sum(exps, axis=-1, keepdims=True)

    h1 = jnp.maximum(
        jnp.dot(x, w1_ref[...], preferred_element_type=jnp.float32)
        + b1_ref[...], 0.0)
    ph = jnp.dot(h1, w2_ref[...], preferred_element_type=jnp.float32) + b2_ref[...]
    ph_norm = ph / jnp.maximum(
        jnp.sqrt(jnp.sum(ph * ph, axis=-1, keepdims=True)), 1e-12)

    sigt = sigt_ref[...]
    sig_norm = sigt / jnp.maximum(
        jnp.sqrt(jnp.sum(sigt * sigt, axis=0, keepdims=True)), 1e-12)
    ph_match = (jnp.dot(ph_norm, sig_norm, preferred_element_type=jnp.float32)
                + 1.0) * 0.5
    out_ref[...] = probs * ph_match


def _routing_scores(x, gate_W, gate_b, ph_W1, ph_b1, ph_W2, ph_b2, sigs):
    full = lambda shape: pl.BlockSpec(shape, lambda i: (0,) * len(shape))
    return pl.pallas_call(
        _routing_scores_body,
        grid=(T // _TT_ROUTE,),
        in_specs=[
            pl.BlockSpec((_TT_ROUTE, D), lambda i: (i, 0)),
            full((D, E)),
            full((1, E)),
            full((D, 32)),
            full((1, 32)),
            full((32, 16)),
            full((1, 16)),
            full((16, E)),
        ],
        out_specs=pl.BlockSpec((_TT_ROUTE, E), lambda i: (i, 0)),
        out_shape=jax.ShapeDtypeStruct((T, E), jnp.float32),
        compiler_params=pltpu.CompilerParams(
            dimension_semantics=("parallel",)),
    )(x, gate_W, gate_b.reshape(1, E), ph_W1, ph_b1.reshape(1, 32), ph_W2,
      ph_b2.reshape(1, 16), sigs.T)


_NC = 2
_NS = 16
_NW = _NC * _NS
_TOK_PER_W = T // _NW
_VALS_PER_W = _TOK_PER_W * E
_LANES = 16
_CHUNKS = _TOK_PER_W // _LANES


def _topk_normalize_body(eff_hbm, out_hbm, vin, vout):
    wid = lax.axis_index("s") * _NC + lax.axis_index("c")
    base = wid * _VALS_PER_W
    pltpu.sync_copy(eff_hbm.at[pl.ds(base, _VALS_PER_W)], vin)

    def chunk(c, carry):
        lane = lax.iota(jnp.int32, _LANES)
        tok = c * _LANES + lane
        idx = [tok * E + e for e in range(E)]
        v = [plsc.load_gather(vin, [idx[e]]) for e in range(E)]

        total = jnp.zeros((_LANES,), jnp.float32)
        w = []
        for e in range(E):
            rank = jnp.zeros((_LANES,), jnp.float32)
            for j in range(E):
                if j == e:
                    continue
                beats = (v[j] >= v[e]) if j < e else (v[j] > v[e])
                rank = rank + jnp.where(beats, 1.0, 0.0)
            we = jnp.where(rank < float(K_ACTIVE), v[e], 0.0)
            total = total + we
            w.append(we)
        inv = 1.0 / (total + 1e-8)
        for e in range(E):
            plsc.store_scatter(vout, [idx[e]], w[e] * inv)
        return carry

    lax.fori_loop(0, _CHUNKS, chunk, 0)
    pltpu.sync_copy(vout, out_hbm.at[pl.ds(base, _VALS_PER_W)])


def _topk_normalize(effective):
    kern = functools.partial(
        pl.kernel,
        out_type=jax.ShapeDtypeStruct((T * E,), jnp.float32),
        mesh=plsc.VectorSubcoreMesh(core_axis_name="c", subcore_axis_name="s"),
        scratch_types=[
            pltpu.VMEM((_VALS_PER_W,), jnp.float32),
            pltpu.VMEM((_VALS_PER_W,), jnp.float32),
        ],
        compiler_params=pltpu.CompilerParams(needs_layout_passes=False),
    )(_topk_normalize_body)
    return kern(effective.reshape(T * E)).reshape(T, E)


_TT_EXP = 1024


def _experts_body(x_ref, w_ref, w1_ref, b1_ref, w2_ref, b2_ref, out_ref):
    x = x_ref[...]
    w = w_ref[...]
    lane = lax.broadcasted_iota(jnp.int32, (_TT_EXP, E), 1)
    acc = None
    for e in range(E):
        h = jnp.maximum(
            jnp.dot(x, w1_ref[e], preferred_element_type=jnp.float32)
            + b1_ref[e, 0], 0.0)
        y = (jnp.dot(h, w2_ref[e], preferred_element_type=jnp.float32)
             + b2_ref[e, 0])
        wsel = jnp.sum(jnp.where(lane == e, w, 0.0), axis=1, keepdims=True)
        contrib = y * wsel
        acc = contrib if acc is None else acc + contrib
    out_ref[...] = acc


def _experts_combine(x, weights, eW1, eb1, eW2, eb2):
    return pl.pallas_call(
        _experts_body,
        grid=(T // _TT_EXP,),
        in_specs=[
            pl.BlockSpec((_TT_EXP, D), lambda i: (i, 0)),
            pl.BlockSpec((_TT_EXP, E), lambda i: (i, 0)),
            pl.BlockSpec((E, D, H), lambda i: (0, 0, 0)),
            pl.BlockSpec((E, 1, H), lambda i: (0, 0, 0)),
            pl.BlockSpec((E, H, O), lambda i: (0, 0, 0)),
            pl.BlockSpec((E, 1, O), lambda i: (0, 0, 0)),
        ],
        out_specs=pl.BlockSpec((_TT_EXP, O), lambda i: (i, 0)),
        out_shape=jax.ShapeDtypeStruct((T, O), jnp.float32),
        compiler_params=pltpu.CompilerParams(
            dimension_semantics=("arbitrary",)),
    )(x, weights, eW1, eb1.reshape(E, 1, H), eW2, eb2.reshape(E, 1, O))


@jax.jit
def kernel(x, gate_W, gate_b, ph_W1, ph_b1, ph_W2, ph_b2, sigs, eW1, eb1,
           eW2, eb2):
    effective = _routing_scores(x, gate_W, gate_b, ph_W1, ph_b1, ph_W2, ph_b2,
                                sigs)
    weights = _topk_normalize(effective)
    return _experts_combine(x, weights, eW1, eb1, eW2, eb2)


# speedup vs baseline: 1.0687x; 1.0003x over previous
import functools

import jax
import jax.numpy as jnp
import numpy as np
from jax import lax
from jax.experimental import pallas as pl
from jax.experimental.pallas import tpu as pltpu
from jax.experimental.pallas import tpu_sc as plsc

T = 8192
D = 768
H = 768
O = 768
E = 8
K_ACTIVE = 5

_INV_TEMP = float(1.0 / np.e)


_TT_ROUTE = 2048


def _routing_scores_body(x_ref, gw_ref, gb_ref, w1_ref, b1_ref, w2_ref, b2_ref,
                         sigt_ref, out_ref):
    x = x_ref[...]
    scores = (jnp.dot(x, gw_ref[...], preferred_element_type=jnp.float32)
              + gb_ref[...]) * _INV_TEMP
    scores = scores - jnp.max(scores, axis=-1, keepdims=True)
    exps = jnp.exp(scores)
    probs = exps / jnp.sum(exps, axis=-1, keepdims=True)

    h1 = jnp.maximum(
        jnp.dot(x, w1_ref[...], preferred_element_type=jnp.float32)
        + b1_ref[...], 0.0)
    ph = jnp.dot(h1, w2_ref[...], preferred_element_type=jnp.float32) + b2_ref[...]
    ph_norm = ph / jnp.maximum(
        jnp.sqrt(jnp.sum(ph * ph, axis=-1, keepdims=True)), 1e-12)

    sigt = sigt_ref[...]
    sig_norm = sigt / jnp.maximum(
        jnp.sqrt(jnp.sum(sigt * sigt, axis=0, keepdims=True)), 1e-12)
    ph_match = (jnp.dot(ph_norm, sig_norm, preferred_element_type=jnp.float32)
                + 1.0) * 0.5
    out_ref[...] = probs * ph_match


def _routing_scores(x, gate_W, gate_b, ph_W1, ph_b1, ph_W2, ph_b2, sigs):
    full = lambda shape: pl.BlockSpec(shape, lambda i: (0,) * len(shape))
    return pl.pallas_call(
        _routing_scores_body,
        grid=(T // _TT_ROUTE,),
        in_specs=[
            pl.BlockSpec((_TT_ROUTE, D), lambda i: (i, 0)),
            full((D, E)),
            full((1, E)),
            full((D, 32)),
            full((1, 32)),
            full((32, 16)),
            full((1, 16)),
            full((16, E)),
        ],
        out_specs=pl.BlockSpec((_TT_ROUTE, E), lambda i: (i, 0)),
        out_shape=jax.ShapeDtypeStruct((T, E), jnp.float32),
        compiler_params=pltpu.CompilerParams(
            dimension_semantics=("parallel",)),
    )(x, gate_W, gate_b.reshape(1, E), ph_W1, ph_b1.reshape(1, 32), ph_W2,
      ph_b2.reshape(1, 16), sigs.T)


_NC = 2
_NS = 16
_NW = _NC * _NS
_TOK_PER_W = T // _NW
_VALS_PER_W = _TOK_PER_W * E
_LANES = 16
_CHUNKS = _TOK_PER_W // _LANES


def _topk_normalize_body(eff_hbm, out_hbm, vin, vout):
    wid = lax.axis_index("s") * _NC + lax.axis_index("c")
    base = wid * _VALS_PER_W
    pltpu.sync_copy(eff_hbm.at[pl.ds(base, _VALS_PER_W)], vin)

    def chunk(c, carry):
        lane = lax.iota(jnp.int32, _LANES)
        tok = c * _LANES + lane
        idx = [tok * E + e for e in range(E)]
        v = [plsc.load_gather(vin, [idx[e]]) for e in range(E)]

        total = jnp.zeros((_LANES,), jnp.float32)
        w = []
        for e in range(E):
            rank = jnp.zeros((_LANES,), jnp.float32)
            for j in range(E):
                if j == e:
                    continue
                beats = (v[j] >= v[e]) if j < e else (v[j] > v[e])
                rank = rank + jnp.where(beats, 1.0, 0.0)
            we = jnp.where(rank < float(K_ACTIVE), v[e], 0.0)
            total = total + we
            w.append(we)
        inv = 1.0 / (total + 1e-8)
        for e in range(E):
            plsc.store_scatter(vout, [idx[e]], w[e] * inv)
        return carry

    lax.fori_loop(0, _CHUNKS, chunk, 0)
    pltpu.sync_copy(vout, out_hbm.at[pl.ds(base, _VALS_PER_W)])


def _topk_normalize(effective):
    kern = functools.partial(
        pl.kernel,
        out_type=jax.ShapeDtypeStruct((T * E,), jnp.float32),
        mesh=plsc.VectorSubcoreMesh(core_axis_name="c", subcore_axis_name="s"),
        scratch_types=[
            pltpu.VMEM((_VALS_PER_W,), jnp.float32),
            pltpu.VMEM((_VALS_PER_W,), jnp.float32),
        ],
        compiler_params=pltpu.CompilerParams(needs_layout_passes=False),
    )(_topk_normalize_body)
    return kern(effective.reshape(T * E)).reshape(T, E)


_TT_EXP = 1024


def _experts_body(x_ref, w_ref, w1_ref, b1_ref, w2_ref, b2_ref, out_ref):
    x = x_ref[...]
    acc = None
    for e in range(E):
        h = jnp.maximum(
            jnp.dot(x, w1_ref[e], preferred_element_type=jnp.float32)
            + b1_ref[e, 0], 0.0)
        y = (jnp.dot(h, w2_ref[e], preferred_element_type=jnp.float32)
             + b2_ref[e, 0])
        contrib = y * w_ref[:, e:e + 1]
        acc = contrib if acc is None else acc + contrib
    out_ref[...] = acc


def _experts_combine(x, weights, eW1, eb1, eW2, eb2):
    return pl.pallas_call(
        _experts_body,
        grid=(T // _TT_EXP,),
        in_specs=[
            pl.BlockSpec((_TT_EXP, D), lambda i: (i, 0)),
            pl.BlockSpec((_TT_EXP, E), lambda i: (i, 0)),
            pl.BlockSpec((E, D, H), lambda i: (0, 0, 0)),
            pl.BlockSpec((E, 1, H), lambda i: (0, 0, 0)),
            pl.BlockSpec((E, H, O), lambda i: (0, 0, 0)),
            pl.BlockSpec((E, 1, O), lambda i: (0, 0, 0)),
        ],
        out_specs=pl.BlockSpec((_TT_EXP, O), lambda i: (i, 0)),
        out_shape=jax.ShapeDtypeStruct((T, O), jnp.float32),
        compiler_params=pltpu.CompilerParams(
            dimension_semantics=("arbitrary",)),
    )(x, weights, eW1, eb1.reshape(E, 1, H), eW2, eb2.reshape(E, 1, O))


@jax.jit
def kernel(x, gate_W, gate_b, ph_W1, ph_b1, ph_W2, ph_b2, sigs, eW1, eb1,
           eW2, eb2):
    effective = _routing_scores(x, gate_W, gate_b, ph_W1, ph_b1, ph_W2, ph_b2,
                                sigs)
    weights = _topk_normalize(effective)
    return _experts_combine(x, weights, eW1, eb1, eW2, eb2)
